# baseline (speedup 1.0000x reference)
import jax
import jax.numpy as jnp
from jax.experimental import pallas as pl
from jax.experimental.pallas import tpu as pltpu

_B, _L, _C, _H = 64, 576, 384, 24
_OVERLAP, _NUM_VQS = 4, 6
_CB_DIM, _CB_SIZE = 256, 8192
_W_SP = _L // _H
_T = _W_SP // _OVERLAP
_FIX_DIM = _H * _C
_GROUP_DIM = (_OVERLAP * _FIX_DIM) // _NUM_VQS
_ROWS = _B * _T
_EPS = 1e-12
_K_TILE = 1024
_NUM_KT = _CB_SIZE // _K_TILE


def _proj_body(x_ref, w_ref, o_ref):
    x = x_ref[...]
    w = w_ref[0]
    ze = jnp.dot(x, w, preferred_element_type=jnp.float32)
    n = jnp.sqrt(jnp.sum(ze * ze, axis=1, keepdims=True))
    o_ref[0] = ze / (n + _EPS)


def _argmax_body(zen_ref, cb_ref, zq_ref, bestv_ref):
    k = pl.program_id(1)
    zen = zen_ref[0]
    tile = cb_ref[0]
    rn = jnp.sqrt(jnp.sum(tile * tile, axis=1, keepdims=True))
    tile_n = tile / (rn + _EPS)
    sim = jax.lax.dot_general(
        zen, tile_n, (((1,), (1,)), ((), ())),
        preferred_element_type=jnp.float32)
    m = jnp.max(sim, axis=1, keepdims=True)
    lane = jax.lax.broadcasted_iota(jnp.int32, (_ROWS, _K_TILE), 1)
    big = jnp.int32(2**30)
    loc = jnp.min(jnp.where(sim == m, lane, big), axis=1, keepdims=True)
    onehot = (lane == loc).astype(jnp.bfloat16)
    hi = tile_n.astype(jnp.bfloat16)
    lo = (tile_n - hi.astype(jnp.float32)).astype(jnp.bfloat16)
    zq_tile = (jnp.dot(onehot, hi, preferred_element_type=jnp.float32)
               + jnp.dot(onehot, lo, preferred_element_type=jnp.float32))

    @pl.when(k == 0)
    def _():
        bestv_ref[...] = m
        zq_ref[0] = zq_tile

    @pl.when(k > 0)
    def _():
        upd = m > bestv_ref[...]
        bestv_ref[...] = jnp.where(upd, m, bestv_ref[...])
        zq_ref[0] = jnp.where(upd, zq_tile, zq_ref[0])


def _out_body(zq_ref, zen_ref, w_ref, o_ref, loss_ref):
    g = pl.program_id(0)
    zqn = zq_ref[0]
    w = w_ref[0]
    o_ref[...] = jnp.dot(zqn, w, preferred_element_type=jnp.float32)
    diff = zen_ref[0] - zqn
    rs = jnp.sum(diff * diff, axis=1)
    lane = jax.lax.broadcasted_iota(jnp.int32, (_B, _ROWS), 1)
    sub = jax.lax.broadcasted_iota(jnp.int32, (_B, _ROWS), 0)
    mask = (lane // _T) == sub
    contrib = jnp.sum(jnp.where(mask, rs[None, :], 0.0), axis=1)
    contrib = contrib * (1.0 / (_T * _CB_DIM * _NUM_VQS))

    @pl.when(g == 0)
    def _():
        loss_ref[0] = contrib

    @pl.when(g > 0)
    def _():
        loss_ref[0] = loss_ref[0] + contrib


def kernel(z, Win, codebooks, Wout):
    z4 = z.reshape(_B, _H, _W_SP, _C)
    zt = jnp.transpose(z4, (0, 2, 3, 1)).reshape(_B, _T, _OVERLAP * _FIX_DIM)
    zp = zt.reshape(_ROWS, _NUM_VQS * _GROUP_DIM)

    zen = pl.pallas_call(
        _proj_body,
        grid=(_NUM_VQS,),
        in_specs=[
            pl.BlockSpec((_ROWS, _GROUP_DIM), lambda g: (0, g)),
            pl.BlockSpec((1, _GROUP_DIM, _CB_DIM), lambda g: (g, 0, 0)),
        ],
        out_specs=pl.BlockSpec((1, _ROWS, _CB_DIM), lambda g: (g, 0, 0)),
        out_shape=jax.ShapeDtypeStruct((_NUM_VQS, _ROWS, _CB_DIM), jnp.float32),
    )(zp, Win)

    zq = pl.pallas_call(
        _argmax_body,
        grid=(_NUM_VQS, _NUM_KT),
        in_specs=[
            pl.BlockSpec((1, _ROWS, _CB_DIM), lambda g, k: (g, 0, 0)),
            pl.BlockSpec((1, _K_TILE, _CB_DIM), lambda g, k: (g, k, 0)),
        ],
        out_specs=pl.BlockSpec((1, _ROWS, _CB_DIM), lambda g, k: (g, 0, 0)),
        out_shape=jax.ShapeDtypeStruct((_NUM_VQS, _ROWS, _CB_DIM), jnp.float32),
        scratch_shapes=[
            pltpu.VMEM((_ROWS, 1), jnp.float32),
        ],
    )(zen, codebooks)

    zq_cols, loss = pl.pallas_call(
        _out_body,
        grid=(_NUM_VQS,),
        in_specs=[
            pl.BlockSpec((1, _ROWS, _CB_DIM), lambda g: (g, 0, 0)),
            pl.BlockSpec((1, _ROWS, _CB_DIM), lambda g: (g, 0, 0)),
            pl.BlockSpec((1, _CB_DIM, _GROUP_DIM), lambda g: (g, 0, 0)),
        ],
        out_specs=[
            pl.BlockSpec((_ROWS, _GROUP_DIM), lambda g: (0, g)),
            pl.BlockSpec((1, _B), lambda g: (0, 0)),
        ],
        out_shape=[
            jax.ShapeDtypeStruct((_ROWS, _NUM_VQS * _GROUP_DIM), jnp.float32),
            jax.ShapeDtypeStruct((1, _B), jnp.float32),
        ],
    )(zq, zen, Wout)

    zq3 = zq_cols.reshape(_B, _T, _OVERLAP * _FIX_DIM)
    zq2 = zq3.reshape(_B, _W_SP, _FIX_DIM).reshape(_B, _W_SP, _C, _H)
    out = jnp.transpose(zq2, (0, 3, 1, 2)).reshape(_B, _L, _C)
    lossv = loss.reshape(_B)
    return out, lossv, lossv


# speedup vs baseline: 1.0391x; 1.0391x over previous
import jax
import jax.numpy as jnp
from jax.experimental import pallas as pl
from jax.experimental.pallas import tpu as pltpu

_B, _L, _C, _H = 64, 576, 384, 24
_OVERLAP, _NUM_VQS = 4, 6
_CB_DIM, _CB_SIZE = 256, 8192
_W_SP = _L // _H
_T = _W_SP // _OVERLAP
_FIX_DIM = _H * _C
_GROUP_DIM = (_OVERLAP * _FIX_DIM) // _NUM_VQS
_ROWS = _B * _T
_EPS = 1e-12
_NUM_KT = 4
_K_TILE = _CB_SIZE // _NUM_KT
_CHUNK = _GROUP_DIM // _NUM_KT
_LOSS_SCALE = 1.0 / (_T * _CB_DIM * _NUM_VQS)


def _body(zp_ref, win_ref, cb_ref, wout_ref, o_ref, loss_ref,
          zen_cur, zen_next, zq_ref, zq_prev, best_ref):
    g = pl.program_id(0)
    k = pl.program_id(1)
    last = _NUM_KT - 1

    @pl.when(g >= 2)
    def _():
        w = wout_ref[0]
        o_ref[...] = jnp.dot(zq_prev[...].astype(jnp.bfloat16), w,
                             preferred_element_type=jnp.float32)

    @pl.when((g >= 1) & (g <= _NUM_VQS))
    def _():
        zen = zen_cur[...]
        tile = cb_ref[0]
        rn = jnp.sqrt(jnp.sum(tile * tile, axis=1, keepdims=True))
        tile_n = tile / (rn + _EPS)
        sim = jax.lax.dot_general(
            zen, tile_n, (((1,), (1,)), ((), ())),
            preferred_element_type=jnp.float32)
        m = jnp.max(sim, axis=1, keepdims=True)
        lane = jax.lax.broadcasted_iota(jnp.int32, (_ROWS, _K_TILE), 1)
        big = jnp.int32(2**30)
        loc = jnp.min(jnp.where(sim == m, lane, big), axis=1, keepdims=True)
        onehot = (lane == loc).astype(jnp.bfloat16)
        zq_tile = jnp.dot(onehot, tile_n.astype(jnp.bfloat16),
                          preferred_element_type=jnp.float32)

        @pl.when(k == 0)
        def _():
            best_ref[...] = m
            zq_ref[...] = zq_tile

        @pl.when(k > 0)
        def _():
            upd = m > best_ref[...]
            best_ref[...] = jnp.where(upd, m, best_ref[...])
            zq_ref[...] = jnp.where(upd, zq_tile, zq_ref[...])

        @pl.when(k == last)
        def _():
            rs = 2.0 - 2.0 * best_ref[...][:, 0]
            lane2 = jax.lax.broadcasted_iota(jnp.int32, (_B, _ROWS), 1)
            sub = jax.lax.broadcasted_iota(jnp.int32, (_B, _ROWS), 0)
            mask = (lane2 // _T) == sub
            contrib = jnp.sum(jnp.where(mask, rs[None, :], 0.0), axis=1)
            contrib = contrib * _LOSS_SCALE

            @pl.when(g == 1)
            def _():
                loss_ref[0] = contrib

            @pl.when(g > 1)
            def _():
                loss_ref[0] = loss_ref[0] + contrib

            zq_prev[...] = zq_ref[...]

    @pl.when(g < _NUM_VQS)
    def _():
        x = zp_ref[...]
        w = win_ref[0]
        prod = jnp.dot(x, w, preferred_element_type=jnp.float32)

        @pl.when(k == 0)
        def _():
            zen_next[...] = prod

        @pl.when(k > 0)
        def _():
            zen_next[...] = zen_next[...] + prod

        @pl.when(k == last)
        def _():
            ze = zen_next[...]
            n = jnp.sqrt(jnp.sum(ze * ze, axis=1, keepdims=True))
            zen_cur[...] = ze / (n + _EPS)


def kernel(z, Win, codebooks, Wout):
    z4 = z.reshape(_B, _H, _W_SP, _C)
    zt = jnp.transpose(z4, (0, 2, 3, 1)).reshape(_B, _T, _OVERLAP * _FIX_DIM)
    zp = zt.reshape(_ROWS, _NUM_VQS * _GROUP_DIM)
    wout_bf16 = Wout.astype(jnp.bfloat16)

    nv, nk, last = _NUM_VQS, _NUM_KT, _NUM_KT - 1

    def zp_idx(g, k):
        return (0, jnp.where(g < nv, g * nk + k, nv * nk - 1))

    def win_idx(g, k):
        return (jnp.minimum(g, nv - 1), jnp.where(g < nv, k, last), 0)

    def cb_idx(g, k):
        gs = jnp.clip(g - 1, 0, nv - 1)
        return (gs, jnp.where((g >= 1) & (g <= nv), k, 0), 0)

    def wout_idx(g, k):
        gb = jnp.clip(g - 2, 0, nv - 1)
        return (gb, 0, jnp.where(g >= 2, k, 0))

    def out_idx(g, k):
        gb = jnp.clip(g - 2, 0, nv - 1)
        return (0, jnp.where(g >= 2, gb * nk + k, 0))

    zq_cols, loss = pl.pallas_call(
        _body,
        grid=(nv + 2, nk),
        in_specs=[
            pl.BlockSpec((_ROWS, _CHUNK), zp_idx),
            pl.BlockSpec((1, _CHUNK, _CB_DIM), win_idx),
            pl.BlockSpec((1, _K_TILE, _CB_DIM), cb_idx),
            pl.BlockSpec((1, _CB_DIM, _CHUNK), wout_idx),
        ],
        out_specs=[
            pl.BlockSpec((_ROWS, _CHUNK), out_idx),
            pl.BlockSpec((1, _B), lambda g, k: (0, 0)),
        ],
        out_shape=[
            jax.ShapeDtypeStruct((_ROWS, _NUM_VQS * _GROUP_DIM), jnp.float32),
            jax.ShapeDtypeStruct((1, _B), jnp.float32),
        ],
        scratch_shapes=[
            pltpu.VMEM((_ROWS, _CB_DIM), jnp.float32),
            pltpu.VMEM((_ROWS, _CB_DIM), jnp.float32),
            pltpu.VMEM((_ROWS, _CB_DIM), jnp.float32),
            pltpu.VMEM((_ROWS, _CB_DIM), jnp.float32),
            pltpu.VMEM((_ROWS, 1), jnp.float32),
        ],
    )(zp, Win, codebooks, wout_bf16)

    zq2 = zq_cols.reshape(_B, _W_SP, _C, _H)
    out = jnp.transpose(zq2, (0, 3, 1, 2)).reshape(_B, _L, _C)
    lossv = loss.reshape(_B)
    return out, lossv, lossv
